# baseline (speedup 1.0000x reference)
import functools

import jax
import jax.numpy as jnp
from jax import lax
from jax.experimental import pallas as pl
from jax.experimental.pallas import tpu as pltpu
from jax.experimental.pallas import tpu_sc as plsc

_N = 10000
_E = 160000
_D = 256
_HID = 256
_OUT = 128
_L = 4

_NC = 2
_NS = 16
_NW = _NC * _NS
_LANE = 16

_NCK = 2
_EC = _E // _NCK

_EW = _EC // _NW
_GC = 64
_GT = -(-(_EW + 8) // _GC)
_GT += _GT % 2
_PREF = _GT * _GC

_SE = _EC // _NS
_SCC = 40
_SCT = _SE // _SCC
_FH = _HID // _NC
_NR = 640
_NPAD = _NR * _NS
_NTAIL = _N - 15 * _NR


def _silu(x):
    return x * jax.nn.sigmoid(x)


def _tc_proj(h, w1a, w1b, bm1):
    bn = 2000

    def body(h_ref, wa_ref, wb_ref, b_ref, a_ref, bo_ref):
        hb = h_ref[...]
        a_ref[...] = jnp.dot(hb, wa_ref[...],
                             preferred_element_type=jnp.float32) + b_ref[...]
        bo_ref[...] = jnp.dot(hb, wb_ref[...],
                              preferred_element_type=jnp.float32)

    return pl.pallas_call(
        body,
        grid=(_N // bn,),
        in_specs=[
            pl.BlockSpec((bn, _D), lambda i: (i, 0)),
            pl.BlockSpec((_D, _HID), lambda i: (0, 0)),
            pl.BlockSpec((_D, _HID), lambda i: (0, 0)),
            pl.BlockSpec((1, _HID), lambda i: (0, 0)),
        ],
        out_specs=[pl.BlockSpec((bn, _HID), lambda i: (i, 0))] * 2,
        out_shape=[jax.ShapeDtypeStruct((_N, _HID), jnp.float32)] * 2,
    )(h, w1a, w1b, bm1.reshape(1, _HID))


def _tc_edge(g, wm2, bm2, wa_row, ba):
    be = 1600

    def body(g_ref, w_ref, b_ref, wa_ref, ba_ref, o_ref):
        m = _silu(g_ref[...])
        m2 = _silu(jnp.dot(m, w_ref[...],
                           preferred_element_type=jnp.float32) + b_ref[...])
        logit = jnp.sum(m2 * wa_ref[...], axis=1, keepdims=True) + ba_ref[0, 0]
        o_ref[...] = m2 * jax.nn.sigmoid(logit)

    return pl.pallas_call(
        body,
        grid=(_EC // be,),
        in_specs=[
            pl.BlockSpec((be, _HID), lambda i: (i, 0)),
            pl.BlockSpec((_HID, _HID), lambda i: (0, 0)),
            pl.BlockSpec((1, _HID), lambda i: (0, 0)),
            pl.BlockSpec((1, _HID), lambda i: (0, 0)),
            pl.BlockSpec((1, 1), lambda i: (0, 0)),
        ],
        out_specs=pl.BlockSpec((be, _HID), lambda i: (i, 0)),
        out_shape=jax.ShapeDtypeStruct((_EC, _HID), jnp.float32),
    )(g, wm2, bm2.reshape(1, _HID), wa_row, ba.reshape(1, 1))


def _tc_node(h, agg1, agg2, cnt1, cnt2, wh1h, wh1m, bh1, wh2, bh2):
    bn = 2000

    def body(h_ref, a1_ref, a2_ref, c1_ref, c2_ref, w1h_ref, w1m_ref, b1_ref,
             w2_ref, b2_ref, o_ref):
        denom = jnp.maximum(c1_ref[:, 0:1] + c2_ref[:, 0:1], 1.0)
        mean = (a1_ref[...] + a2_ref[...]) / denom
        z = _silu(
            jnp.dot(h_ref[...], w1h_ref[...],
                    preferred_element_type=jnp.float32)
            + jnp.dot(mean, w1m_ref[...], preferred_element_type=jnp.float32)
            + b1_ref[...])
        o_ref[...] = jnp.dot(z, w2_ref[...],
                             preferred_element_type=jnp.float32) + b2_ref[...]

    return pl.pallas_call(
        body,
        grid=(_N // bn,),
        in_specs=[
            pl.BlockSpec((bn, _D), lambda i: (i, 0)),
            pl.BlockSpec((bn, _HID), lambda i: (i, 0)),
            pl.BlockSpec((bn, _HID), lambda i: (i, 0)),
            pl.BlockSpec((bn, _FH), lambda i: (i, 0)),
            pl.BlockSpec((bn, _FH), lambda i: (i, 0)),
            pl.BlockSpec((_D, _HID), lambda i: (0, 0)),
            pl.BlockSpec((_HID, _HID), lambda i: (0, 0)),
            pl.BlockSpec((1, _HID), lambda i: (0, 0)),
            pl.BlockSpec((_HID, _HID), lambda i: (0, 0)),
            pl.BlockSpec((1, _HID), lambda i: (0, 0)),
        ],
        out_specs=pl.BlockSpec((bn, _HID), lambda i: (i, 0)),
        out_shape=jax.ShapeDtypeStruct((_N, _HID), jnp.float32),
    )(h, agg1, agg2, cnt1, cnt2, wh1h, wh1m, bh1.reshape(1, _HID), wh2,
      bh2.reshape(1, _HID))


def _tc_final(h, wo1, bo1, wo2, bo2, wo3, bo3):
    bn = 1000

    def body(h_ref, w1_ref, b1_ref, w2_ref, b2_ref, w3_ref, b3_ref, o_ref):
        t = _silu(jnp.dot(h_ref[...], w1_ref[...],
                          preferred_element_type=jnp.float32) + b1_ref[...])
        t = jax.nn.relu(jnp.dot(t, w2_ref[...],
                                preferred_element_type=jnp.float32)
                        + b2_ref[...])
        o_ref[...] = jnp.dot(t, w3_ref[...],
                             preferred_element_type=jnp.float32) + b3_ref[...]

    return pl.pallas_call(
        body,
        grid=(_N // bn,),
        in_specs=[
            pl.BlockSpec((bn, _HID), lambda i: (i, 0)),
            pl.BlockSpec((_HID, 1024), lambda i: (0, 0)),
            pl.BlockSpec((1, 1024), lambda i: (0, 0)),
            pl.BlockSpec((1024, 1024), lambda i: (0, 0)),
            pl.BlockSpec((1, 1024), lambda i: (0, 0)),
            pl.BlockSpec((1024, _OUT), lambda i: (0, 0)),
            pl.BlockSpec((1, _OUT), lambda i: (0, 0)),
        ],
        out_specs=pl.BlockSpec((bn, _OUT), lambda i: (i, 0)),
        out_shape=jax.ShapeDtypeStruct((_N, _OUT), jnp.float32),
    )(h, wo1, bo1.reshape(1, 1024), wo2, bo2.reshape(1, 1024), wo3,
      bo3.reshape(1, _OUT))


def _ring(n_chunks, start, process):
    start(0, 0)
    start(1, 1)

    def pair(p, carry):
        for s in (0, 1):
            t = 2 * p + s

            @pl.when(t < n_chunks)
            def _():
                process(t, s)

                @pl.when(t + 2 < n_chunks)
                def _():
                    start(t + 2, s)

        return carry

    lax.fori_loop(0, (n_chunks + 1) // 2, pair, 0)


def _sc_gather(a, b, idx_i, idx_j):
    mesh = plsc.VectorSubcoreMesh(core_axis_name="c", subcore_axis_name="s")

    @functools.partial(
        pl.kernel,
        out_type=jax.ShapeDtypeStruct((_EC, _HID), jnp.float32),
        mesh=mesh,
        scratch_types=[
            pltpu.VMEM((_PREF,), jnp.int32),
            pltpu.VMEM((_PREF,), jnp.int32),
            pltpu.VMEM((2, _GC, _HID), jnp.float32),
            pltpu.VMEM((2, _GC, _HID), jnp.float32),
            pltpu.VMEM((2, _GC, _HID), jnp.float32),
            pltpu.SemaphoreType.DMA,
            pltpu.SemaphoreType.DMA,
            pltpu.SemaphoreType.DMA,
            pltpu.SemaphoreType.DMA,
        ],
    )
    def k(a_hbm, b_hbm, ii_hbm, jj_hbm, g_hbm, ii_v, jj_v, ra_v, rb_v, ob_v,
          gsem0, gsem1, osem0, osem1):
        cid = lax.axis_index("c")
        sid = lax.axis_index("s")
        wid = sid * _NC + cid
        gsems = (gsem0, gsem1)
        osems = (osem0, osem1)

        base = (wid * _EW) // 8 * 8
        limit = jnp.where(wid == _NW - 1, _EC, ((wid + 1) * _EW) // 8 * 8)
        pref0 = jnp.minimum(base, _EC - _PREF)

        pltpu.sync_copy(ii_hbm.at[pl.ds(pref0, _PREF)], ii_v)
        pltpu.sync_copy(jj_hbm.at[pl.ds(pref0, _PREF)], jj_v)

        def off_of(t):
            return jnp.minimum(base + t * _GC, limit - _GC)

        def start(t, s):
            loff = off_of(t) - pref0
            pltpu.async_copy(a_hbm.at[ii_v.at[pl.ds(loff, _GC)]], ra_v.at[s],
                             gsems[s])
            pltpu.async_copy(b_hbm.at[jj_v.at[pl.ds(loff, _GC)]], rb_v.at[s],
                             gsems[s])

        def wait_out(t, s):
            pltpu.make_async_copy(
                ob_v.at[s], g_hbm.at[pl.ds(off_of(t), _GC)],
                osems[s]).wait()

        def process(t, s):
            pltpu.make_async_copy(a_hbm.at[pl.ds(0, _GC)], ra_v.at[s],
                                  gsems[s]).wait()
            pltpu.make_async_copy(b_hbm.at[pl.ds(0, _GC)], rb_v.at[s],
                                  gsems[s]).wait()

            @pl.when(t >= 2)
            def _():
                wait_out(t - 2, s)

            def addrow(e, carry):
                for kk in range(_HID // _LANE):
                    sl = pl.ds(kk * _LANE, _LANE)
                    ob_v[s, e, sl] = ra_v[s, e, sl] + rb_v[s, e, sl]
                return carry

            lax.fori_loop(0, _GC, addrow, 0)
            pltpu.async_copy(ob_v.at[s], g_hbm.at[pl.ds(off_of(t), _GC)],
                             osems[s])

        _ring(_GT, start, process)
        wait_out(_GT - 2, 0)
        wait_out(_GT - 1, 1)

    return k(a, b, idx_i, idx_j)


def _zero_block(z_v, rows):
    def zrow(r, carry):
        zero = jnp.zeros((_LANE,), jnp.float32)
        for kk in range(_FH // _LANE):
            z_v[r, pl.ds(kk * _LANE, _LANE)] = zero
        return carry

    lax.fori_loop(0, rows, zrow, 0)


def _sc_scatter(msg, idx_i):
    mesh = plsc.VectorSubcoreMesh(core_axis_name="c", subcore_axis_name="s")

    @functools.partial(
        pl.kernel,
        out_type=jax.ShapeDtypeStruct((_N, _HID), jnp.float32),
        mesh=mesh,
        scratch_types=[
            pltpu.VMEM((2, _SCC), jnp.int32),
            pltpu.VMEM((2, _SCC, _FH), jnp.float32),
            pltpu.VMEM((_SCC, _FH), jnp.float32),
            pltpu.VMEM_SHARED((_NPAD, _FH), jnp.float32),
            pltpu.SemaphoreType.DMA,
            pltpu.SemaphoreType.DMA,
        ],
    )
    def k(msg_hbm, ii_hbm, agg_hbm, ii_v, mb_v, z_v, acc_sh, sem0, sem1):
        cid = lax.axis_index("c")
        sid = lax.axis_index("s")
        base = sid * _SE
        col0 = cid * _FH
        sems = (sem0, sem1)
        row0 = sid * _NR

        _zero_block(z_v, _SCC)
        for zc in range(_NR // _SCC):
            pltpu.sync_copy(z_v, acc_sh.at[pl.ds(row0 + zc * _SCC, _SCC)])
        plsc.subcore_barrier()

        def start(t, s):
            off = base + t * _SCC
            pltpu.async_copy(ii_hbm.at[pl.ds(off, _SCC)], ii_v.at[s], sems[s])
            pltpu.async_copy(
                msg_hbm.at[pl.ds(off, _SCC), pl.ds(col0, _FH)], mb_v.at[s],
                sems[s])

        def process(t, s):
            pltpu.make_async_copy(ii_hbm.at[pl.ds(0, _SCC)], ii_v.at[s],
                                  sems[s]).wait()
            pltpu.make_async_copy(
                msg_hbm.at[pl.ds(base, _SCC), pl.ds(col0, _FH)],
                mb_v.at[s], sems[s]).wait()
            pltpu.sync_copy(mb_v.at[s], acc_sh.at[ii_v.at[s]], add=True)

        _ring(_SCT, start, process)
        plsc.subcore_barrier()

        @pl.when(sid < _NS - 1)
        def _():
            pltpu.sync_copy(acc_sh.at[pl.ds(row0, _NR)],
                            agg_hbm.at[pl.ds(row0, _NR), pl.ds(col0, _FH)])

        @pl.when(sid == _NS - 1)
        def _():
            pltpu.sync_copy(
                acc_sh.at[pl.ds(row0, _NTAIL)],
                agg_hbm.at[pl.ds(row0, _NTAIL), pl.ds(col0, _FH)])

    return k(msg, idx_i)


def kernel(h, edge_index, Wm1, bm1, Wm2, bm2, Wa, ba, Wh1, bh1, Wh2, bh2,
           Wo1, bo1, Wo2, bo2, Wo3, bo3):
    idx_i = edge_index[0]
    idx_j = edge_index[1]
    ii = [idx_i[:_EC], idx_i[_EC:]]
    jj = [idx_j[:_EC], idx_j[_EC:]]
    ones = jnp.ones((_EC, _HID), jnp.float32)
    cnt1 = _sc_scatter(ones, ii[0])
    cnt2 = _sc_scatter(ones, ii[1])
    for l in range(_L):
        a, b = _tc_proj(h, Wm1[l, :_D], Wm1[l, _D:], bm1[l])
        wa_row = Wa[l].reshape(1, _HID)
        g1 = _sc_gather(a, b, ii[0], jj[0])
        msg1 = _tc_edge(g1, Wm2[l], bm2[l], wa_row, ba[l])
        g2 = _sc_gather(a, b, ii[1], jj[1])
        msg2 = _tc_edge(g2, Wm2[l], bm2[l], wa_row, ba[l])
        agg1 = _sc_scatter(msg1, ii[0])
        agg2 = _sc_scatter(msg2, ii[1])
        h = _tc_node(h, agg1, agg2, cnt1, cnt2, Wh1[l, :_D], Wh1[l, _D:],
                     bh1[l], Wh2[l], bh2[l])
    return _tc_final(h, Wo1, bo1, Wo2, bo2, Wo3, bo3)


# speedup vs baseline: 1.1456x; 1.1456x over previous
import functools

import jax
import jax.numpy as jnp
from jax import lax
from jax.experimental import pallas as pl
from jax.experimental.pallas import tpu as pltpu
from jax.experimental.pallas import tpu_sc as plsc

_N = 10000
_E = 160000
_D = 256
_HID = 256
_OUT = 128
_L = 4

_NC = 2
_NS = 16
_NW = _NC * _NS
_LANE = 16

_NCK = 2
_EC = _E // _NCK

_EW = _EC // _NW
_GC = 64
_GT = -(-(_EW + 8) // _GC)
_GT += _GT % 2
_PREF = _GT * _GC

_SCC = 128
_SNCH = _EC // _SCC
_FH = _HID // _NC
_NR = 632
_NPAD = _NR * _NS
_NTAIL = _N - 15 * _NR


def _silu(x):
    return x * jax.nn.sigmoid(x)


def _tc_proj(h, w1a, w1b, bm1):
    bn = 2000

    def body(h_ref, wa_ref, wb_ref, b_ref, a_ref, bo_ref):
        hb = h_ref[...]
        a_ref[...] = jnp.dot(hb, wa_ref[...],
                             preferred_element_type=jnp.float32) + b_ref[...]
        bo_ref[...] = jnp.dot(hb, wb_ref[...],
                              preferred_element_type=jnp.float32)

    return pl.pallas_call(
        body,
        grid=(_N // bn,),
        in_specs=[
            pl.BlockSpec((bn, _D), lambda i: (i, 0)),
            pl.BlockSpec((_D, _HID), lambda i: (0, 0)),
            pl.BlockSpec((_D, _HID), lambda i: (0, 0)),
            pl.BlockSpec((1, _HID), lambda i: (0, 0)),
        ],
        out_specs=[pl.BlockSpec((bn, _HID), lambda i: (i, 0))] * 2,
        out_shape=[jax.ShapeDtypeStruct((_N, _HID), jnp.float32)] * 2,
    )(h, w1a, w1b, bm1.reshape(1, _HID))


def _tc_edge(g, wm2, bm2, wa_row, ba):
    be = 1600

    def body(g_ref, w_ref, b_ref, wa_ref, ba_ref, o_ref):
        m = _silu(g_ref[...])
        m2 = _silu(jnp.dot(m, w_ref[...],
                           preferred_element_type=jnp.float32) + b_ref[...])
        logit = jnp.sum(m2 * wa_ref[...], axis=1, keepdims=True) + ba_ref[0, 0]
        o_ref[...] = m2 * jax.nn.sigmoid(logit)

    return pl.pallas_call(
        body,
        grid=(_EC // be,),
        in_specs=[
            pl.BlockSpec((be, _HID), lambda i: (i, 0)),
            pl.BlockSpec((_HID, _HID), lambda i: (0, 0)),
            pl.BlockSpec((1, _HID), lambda i: (0, 0)),
            pl.BlockSpec((1, _HID), lambda i: (0, 0)),
            pl.BlockSpec((1, 1), lambda i: (0, 0)),
        ],
        out_specs=pl.BlockSpec((be, _HID), lambda i: (i, 0)),
        out_shape=jax.ShapeDtypeStruct((_EC, _HID), jnp.float32),
    )(g, wm2, bm2.reshape(1, _HID), wa_row, ba.reshape(1, 1))


def _tc_node(h, agg1, agg2, cnt1, cnt2, wh1h, wh1m, bh1, wh2, bh2):
    bn = 2000

    def body(h_ref, a1_ref, a2_ref, c1_ref, c2_ref, w1h_ref, w1m_ref, b1_ref,
             w2_ref, b2_ref, o_ref):
        denom = jnp.maximum(c1_ref[:, 0:1] + c2_ref[:, 0:1], 1.0)
        mean = (a1_ref[...] + a2_ref[...]) / denom
        z = _silu(
            jnp.dot(h_ref[...], w1h_ref[...],
                    preferred_element_type=jnp.float32)
            + jnp.dot(mean, w1m_ref[...], preferred_element_type=jnp.float32)
            + b1_ref[...])
        o_ref[...] = jnp.dot(z, w2_ref[...],
                             preferred_element_type=jnp.float32) + b2_ref[...]

    return pl.pallas_call(
        body,
        grid=(_N // bn,),
        in_specs=[
            pl.BlockSpec((bn, _D), lambda i: (i, 0)),
            pl.BlockSpec((bn, _HID), lambda i: (i, 0)),
            pl.BlockSpec((bn, _HID), lambda i: (i, 0)),
            pl.BlockSpec((bn, _FH), lambda i: (i, 0)),
            pl.BlockSpec((bn, _FH), lambda i: (i, 0)),
            pl.BlockSpec((_D, _HID), lambda i: (0, 0)),
            pl.BlockSpec((_HID, _HID), lambda i: (0, 0)),
            pl.BlockSpec((1, _HID), lambda i: (0, 0)),
            pl.BlockSpec((_HID, _HID), lambda i: (0, 0)),
            pl.BlockSpec((1, _HID), lambda i: (0, 0)),
        ],
        out_specs=pl.BlockSpec((bn, _HID), lambda i: (i, 0)),
        out_shape=jax.ShapeDtypeStruct((_N, _HID), jnp.float32),
    )(h, agg1, agg2, cnt1, cnt2, wh1h, wh1m, bh1.reshape(1, _HID), wh2,
      bh2.reshape(1, _HID))


def _tc_final(h, wo1, bo1, wo2, bo2, wo3, bo3):
    bn = 1000

    def body(h_ref, w1_ref, b1_ref, w2_ref, b2_ref, w3_ref, b3_ref, o_ref):
        t = _silu(jnp.dot(h_ref[...], w1_ref[...],
                          preferred_element_type=jnp.float32) + b1_ref[...])
        t = jax.nn.relu(jnp.dot(t, w2_ref[...],
                                preferred_element_type=jnp.float32)
                        + b2_ref[...])
        o_ref[...] = jnp.dot(t, w3_ref[...],
                             preferred_element_type=jnp.float32) + b3_ref[...]

    return pl.pallas_call(
        body,
        grid=(_N // bn,),
        in_specs=[
            pl.BlockSpec((bn, _HID), lambda i: (i, 0)),
            pl.BlockSpec((_HID, 1024), lambda i: (0, 0)),
            pl.BlockSpec((1, 1024), lambda i: (0, 0)),
            pl.BlockSpec((1024, 1024), lambda i: (0, 0)),
            pl.BlockSpec((1, 1024), lambda i: (0, 0)),
            pl.BlockSpec((1024, _OUT), lambda i: (0, 0)),
            pl.BlockSpec((1, _OUT), lambda i: (0, 0)),
        ],
        out_specs=pl.BlockSpec((bn, _OUT), lambda i: (i, 0)),
        out_shape=jax.ShapeDtypeStruct((_N, _OUT), jnp.float32),
    )(h, wo1, bo1.reshape(1, 1024), wo2, bo2.reshape(1, 1024), wo3,
      bo3.reshape(1, _OUT))


def _ring(n_chunks, start, process):
    start(0, 0)
    start(1, 1)

    def pair(p, carry):
        for s in (0, 1):
            t = 2 * p + s

            @pl.when(t < n_chunks)
            def _():
                process(t, s)

                @pl.when(t + 2 < n_chunks)
                def _():
                    start(t + 2, s)

        return carry

    lax.fori_loop(0, (n_chunks + 1) // 2, pair, 0)


def _sc_gather(a, b, idx_i, idx_j):
    mesh = plsc.VectorSubcoreMesh(core_axis_name="c", subcore_axis_name="s")

    @functools.partial(
        pl.kernel,
        out_type=jax.ShapeDtypeStruct((_EC, _HID), jnp.float32),
        mesh=mesh,
        scratch_types=[
            pltpu.VMEM((_PREF,), jnp.int32),
            pltpu.VMEM((_PREF,), jnp.int32),
            pltpu.VMEM((2, _GC, _HID), jnp.float32),
            pltpu.VMEM((2, _GC, _HID), jnp.float32),
            pltpu.VMEM((2, _GC, _HID), jnp.float32),
            pltpu.SemaphoreType.DMA,
            pltpu.SemaphoreType.DMA,
            pltpu.SemaphoreType.DMA,
            pltpu.SemaphoreType.DMA,
        ],
    )
    def k(a_hbm, b_hbm, ii_hbm, jj_hbm, g_hbm, ii_v, jj_v, ra_v, rb_v, ob_v,
          gsem0, gsem1, osem0, osem1):
        cid = lax.axis_index("c")
        sid = lax.axis_index("s")
        wid = sid * _NC + cid
        gsems = (gsem0, gsem1)
        osems = (osem0, osem1)

        base = (wid * _EW) // 8 * 8
        limit = jnp.where(wid == _NW - 1, _EC, ((wid + 1) * _EW) // 8 * 8)
        pref0 = jnp.minimum(base, _EC - _PREF)

        pltpu.sync_copy(ii_hbm.at[pl.ds(pref0, _PREF)], ii_v)
        pltpu.sync_copy(jj_hbm.at[pl.ds(pref0, _PREF)], jj_v)

        def off_of(t):
            return jnp.minimum(base + t * _GC, limit - _GC)

        def start(t, s):
            loff = off_of(t) - pref0
            pltpu.async_copy(a_hbm.at[ii_v.at[pl.ds(loff, _GC)]], ra_v.at[s],
                             gsems[s])
            pltpu.async_copy(b_hbm.at[jj_v.at[pl.ds(loff, _GC)]], rb_v.at[s],
                             gsems[s])

        def wait_out(t, s):
            pltpu.make_async_copy(
                ob_v.at[s], g_hbm.at[pl.ds(off_of(t), _GC)],
                osems[s]).wait()

        def process(t, s):
            pltpu.make_async_copy(a_hbm.at[pl.ds(0, _GC)], ra_v.at[s],
                                  gsems[s]).wait()
            pltpu.make_async_copy(b_hbm.at[pl.ds(0, _GC)], rb_v.at[s],
                                  gsems[s]).wait()

            @pl.when(t >= 2)
            def _():
                wait_out(t - 2, s)

            def addrow(e, carry):
                for kk in range(_HID // _LANE):
                    sl = pl.ds(kk * _LANE, _LANE)
                    ob_v[s, e, sl] = ra_v[s, e, sl] + rb_v[s, e, sl]
                return carry

            lax.fori_loop(0, _GC, addrow, 0)
            pltpu.async_copy(ob_v.at[s], g_hbm.at[pl.ds(off_of(t), _GC)],
                             osems[s])

        _ring(_GT, start, process)
        wait_out(_GT - 2, 0)
        wait_out(_GT - 1, 1)

    return k(a, b, idx_i, idx_j)


def _zero_block(z_v, rows):
    def zrow(r, carry):
        zero = jnp.zeros((_LANE,), jnp.float32)
        for kk in range(_FH // _LANE):
            z_v[r, pl.ds(kk * _LANE, _LANE)] = zero
        return carry

    lax.fori_loop(0, rows, zrow, 0)


def _sc_scatter(msg, idx_i, ones_mode):
    mesh = plsc.VectorSubcoreMesh(core_axis_name="c", subcore_axis_name="s")

    @functools.partial(
        pl.kernel,
        out_type=jax.ShapeDtypeStruct((_N, _HID), jnp.float32),
        mesh=mesh,
        scratch_types=[
            pltpu.VMEM((_LANE,), jnp.int32),
            pltpu.VMEM((2, _SCC), jnp.int32),
            pltpu.VMEM((2, _SCC, _FH), jnp.float32),
            pltpu.VMEM((_SCC, _FH), jnp.float32),
            pltpu.VMEM_SHARED((_NPAD, _FH), jnp.float32),
            pltpu.SemaphoreType.DMA,
            pltpu.SemaphoreType.DMA,
        ],
    )
    def k(msg_hbm, ii_hbm, fl_hbm, agg_hbm, fl_v, ii_v, mb_v, z_v, acc_sh,
          sem0, sem1):
        cid = lax.axis_index("c")
        sid = lax.axis_index("s")
        col0 = cid * _FH
        sems = (sem0, sem1)
        row0 = sid * _NR
        c0 = sid * _SNCH // _NS
        c1 = (sid + 1) * _SNCH // _NS
        ntrip = c1 - c0

        pltpu.sync_copy(fl_hbm, fl_v)
        ones = fl_v[pl.ds(0, _LANE)][0] == 1

        _zero_block(z_v, _SCC)
        for zc in range(_NR // _SCC):
            pltpu.sync_copy(z_v, acc_sh.at[pl.ds(row0 + zc * _SCC, _SCC)])
        zrem = _NR - (_NR // _SCC) * _SCC
        if zrem:
            pltpu.sync_copy(
                z_v.at[pl.ds(0, zrem)],
                acc_sh.at[pl.ds(row0 + _NR - zrem, zrem)])
        plsc.subcore_barrier()

        @pl.when(ones)
        def _():
            def orow(r, carry):
                one = jnp.ones((_LANE,), jnp.float32)
                for kk in range(_FH // _LANE):
                    z_v[r, pl.ds(kk * _LANE, _LANE)] = one
                return carry

            lax.fori_loop(0, _SCC, orow, 0)

        def start(t, s):
            off = (c0 + t) * _SCC
            pltpu.async_copy(ii_hbm.at[pl.ds(off, _SCC)], ii_v.at[s], sems[s])

            @pl.when(jnp.logical_not(ones))
            def _():
                pltpu.async_copy(
                    msg_hbm.at[pl.ds(off, _SCC), pl.ds(col0, _FH)],
                    mb_v.at[s], sems[s])

        def process(t, s):
            pltpu.make_async_copy(ii_hbm.at[pl.ds(0, _SCC)], ii_v.at[s],
                                  sems[s]).wait()

            @pl.when(jnp.logical_not(ones))
            def _():
                pltpu.make_async_copy(
                    msg_hbm.at[pl.ds(0, _SCC), pl.ds(col0, _FH)],
                    mb_v.at[s], sems[s]).wait()
                pltpu.sync_copy(mb_v.at[s], acc_sh.at[ii_v.at[s]], add=True)

            @pl.when(ones)
            def _():
                pltpu.sync_copy(z_v, acc_sh.at[ii_v.at[s]], add=True)

        _ring(ntrip, start, process)
        plsc.subcore_barrier()

        @pl.when(sid < _NS - 1)
        def _():
            pltpu.sync_copy(acc_sh.at[pl.ds(row0, _NR)],
                            agg_hbm.at[pl.ds(row0, _NR), pl.ds(col0, _FH)])

        @pl.when(sid == _NS - 1)
        def _():
            pltpu.sync_copy(
                acc_sh.at[pl.ds(row0, _NTAIL)],
                agg_hbm.at[pl.ds(row0, _NTAIL), pl.ds(col0, _FH)])

    flag = jnp.full((_LANE,), 1 if ones_mode else 0, jnp.int32)
    return k(msg, idx_i, flag)


def kernel(h, edge_index, Wm1, bm1, Wm2, bm2, Wa, ba, Wh1, bh1, Wh2, bh2,
           Wo1, bo1, Wo2, bo2, Wo3, bo3):
    idx_i = edge_index[0]
    idx_j = edge_index[1]
    ii = [idx_i[:_EC], idx_i[_EC:]]
    jj = [idx_j[:_EC], idx_j[_EC:]]
    cnt1 = cnt2 = None
    for l in range(_L):
        a, b = _tc_proj(h, Wm1[l, :_D], Wm1[l, _D:], bm1[l])
        wa_row = Wa[l].reshape(1, _HID)
        g1 = _sc_gather(a, b, ii[0], jj[0])
        msg1 = _tc_edge(g1, Wm2[l], bm2[l], wa_row, ba[l])
        g2 = _sc_gather(a, b, ii[1], jj[1])
        msg2 = _tc_edge(g2, Wm2[l], bm2[l], wa_row, ba[l])
        if l == 0:
            cnt1 = _sc_scatter(msg1, ii[0], ones_mode=True)
            cnt2 = _sc_scatter(msg2, ii[1], ones_mode=True)
        agg1 = _sc_scatter(msg1, ii[0], ones_mode=False)
        agg2 = _sc_scatter(msg2, ii[1], ones_mode=False)
        h = _tc_node(h, agg1, agg2, cnt1, cnt2, Wh1[l, :_D], Wh1[l, _D:],
                     bh1[l], Wh2[l], bh2[l])
    return _tc_final(h, Wo1, bo1, Wo2, bo2, Wo3, bo3)


# speedup vs baseline: 1.2723x; 1.1106x over previous
import functools

import jax
import jax.numpy as jnp
from jax import lax
from jax.experimental import pallas as pl
from jax.experimental.pallas import tpu as pltpu
from jax.experimental.pallas import tpu_sc as plsc

_N = 10000
_E = 160000
_D = 256
_HID = 256
_OUT = 128
_L = 4

_NC = 2
_NS = 16
_NW = _NC * _NS
_LANE = 16

_NCK = 2
_EC = _E // _NCK

_EW = _EC // _NW
_GC = 128
_GT = -(-(_EW + 8) // _GC)
_GT += _GT % 2
_PREF = _GT * _GC
_PH = _HID // 2

_SCC = 128
_SNCH = _EC // _SCC
_FH = _HID // _NC
_NR = 632
_NPAD = _NR * _NS
_NTAIL = _N - 15 * _NR


def _silu(x):
    return x * jax.nn.sigmoid(x)


def _pack_pair(even, odd):
    lo = lax.convert_element_type(
        lax.bitcast_convert_type(even.astype(jnp.bfloat16), jnp.uint16),
        jnp.uint32)
    hi = lax.convert_element_type(
        lax.bitcast_convert_type(odd.astype(jnp.bfloat16), jnp.uint16),
        jnp.uint32)
    return lax.bitcast_convert_type(lo | (hi << 16), jnp.float32)


def _unpack_pair(packed):
    u = lax.bitcast_convert_type(packed, jnp.uint32)
    even = lax.bitcast_convert_type(
        lax.convert_element_type(u & 0xFFFF, jnp.uint16), jnp.bfloat16)
    odd = lax.bitcast_convert_type(
        lax.convert_element_type(u >> 16, jnp.uint16), jnp.bfloat16)
    return (lax.convert_element_type(even, jnp.float32),
            lax.convert_element_type(odd, jnp.float32))


def _tc_proj(h, w1ae, w1ao, w1be, w1bo, bm1e, bm1o):
    bn = 2000

    def body(h_ref, wae_ref, wao_ref, wbe_ref, wbo_ref, be_ref, bo_ref,
             a_ref, b_ref):
        hb = h_ref[...]
        ae = jnp.dot(hb, wae_ref[...],
                     preferred_element_type=jnp.float32) + be_ref[...]
        ao = jnp.dot(hb, wao_ref[...],
                     preferred_element_type=jnp.float32) + bo_ref[...]
        a_ref[...] = _pack_pair(ae, ao)
        bbe = jnp.dot(hb, wbe_ref[...], preferred_element_type=jnp.float32)
        bbo = jnp.dot(hb, wbo_ref[...], preferred_element_type=jnp.float32)
        b_ref[...] = _pack_pair(bbe, bbo)

    return pl.pallas_call(
        body,
        grid=(_N // bn,),
        in_specs=[
            pl.BlockSpec((bn, _D), lambda i: (i, 0)),
            pl.BlockSpec((_D, _PH), lambda i: (0, 0)),
            pl.BlockSpec((_D, _PH), lambda i: (0, 0)),
            pl.BlockSpec((_D, _PH), lambda i: (0, 0)),
            pl.BlockSpec((_D, _PH), lambda i: (0, 0)),
            pl.BlockSpec((1, _PH), lambda i: (0, 0)),
            pl.BlockSpec((1, _PH), lambda i: (0, 0)),
        ],
        out_specs=[pl.BlockSpec((bn, _PH), lambda i: (i, 0))] * 2,
        out_shape=[jax.ShapeDtypeStruct((_N, _PH), jnp.float32)] * 2,
    )(h, w1ae, w1ao, w1be, w1bo, bm1e.reshape(1, _PH), bm1o.reshape(1, _PH))


def _tc_edge(g, wm2, bm2, wa_row, ba):
    be = 1600

    def body(g_ref, w_ref, b_ref, wa_ref, ba_ref, o_ref):
        ae, ao = _unpack_pair(g_ref[:, :_PH])
        be_, bo_ = _unpack_pair(g_ref[:, _PH:])
        m = _silu(jnp.concatenate([ae + be_, ao + bo_], axis=1))
        m2 = _silu(jnp.dot(m, w_ref[...],
                           preferred_element_type=jnp.float32) + b_ref[...])
        logit = jnp.sum(m2 * wa_ref[...], axis=1, keepdims=True) + ba_ref[0, 0]
        o_ref[...] = m2 * jax.nn.sigmoid(logit)

    return pl.pallas_call(
        body,
        grid=(_EC // be,),
        in_specs=[
            pl.BlockSpec((be, _HID), lambda i: (i, 0)),
            pl.BlockSpec((_HID, _HID), lambda i: (0, 0)),
            pl.BlockSpec((1, _HID), lambda i: (0, 0)),
            pl.BlockSpec((1, _HID), lambda i: (0, 0)),
            pl.BlockSpec((1, 1), lambda i: (0, 0)),
        ],
        out_specs=pl.BlockSpec((be, _HID), lambda i: (i, 0)),
        out_shape=jax.ShapeDtypeStruct((_EC, _HID), jnp.float32),
    )(g, wm2, bm2.reshape(1, _HID), wa_row, ba.reshape(1, 1))


def _tc_node(h, agg1, agg2, cnt1, cnt2, wh1h, wh1m, bh1, wh2, bh2):
    bn = 2000

    def body(h_ref, a1_ref, a2_ref, c1_ref, c2_ref, w1h_ref, w1m_ref, b1_ref,
             w2_ref, b2_ref, o_ref):
        denom = jnp.maximum(c1_ref[:, 0:1] + c2_ref[:, 0:1], 1.0)
        mean = (a1_ref[...] + a2_ref[...]) / denom
        z = _silu(
            jnp.dot(h_ref[...], w1h_ref[...],
                    preferred_element_type=jnp.float32)
            + jnp.dot(mean, w1m_ref[...], preferred_element_type=jnp.float32)
            + b1_ref[...])
        o_ref[...] = jnp.dot(z, w2_ref[...],
                             preferred_element_type=jnp.float32) + b2_ref[...]

    return pl.pallas_call(
        body,
        grid=(_N // bn,),
        in_specs=[
            pl.BlockSpec((bn, _D), lambda i: (i, 0)),
            pl.BlockSpec((bn, _HID), lambda i: (i, 0)),
            pl.BlockSpec((bn, _HID), lambda i: (i, 0)),
            pl.BlockSpec((bn, _FH), lambda i: (i, 0)),
            pl.BlockSpec((bn, _FH), lambda i: (i, 0)),
            pl.BlockSpec((_D, _HID), lambda i: (0, 0)),
            pl.BlockSpec((_HID, _HID), lambda i: (0, 0)),
            pl.BlockSpec((1, _HID), lambda i: (0, 0)),
            pl.BlockSpec((_HID, _HID), lambda i: (0, 0)),
            pl.BlockSpec((1, _HID), lambda i: (0, 0)),
        ],
        out_specs=pl.BlockSpec((bn, _HID), lambda i: (i, 0)),
        out_shape=jax.ShapeDtypeStruct((_N, _HID), jnp.float32),
    )(h, agg1, agg2, cnt1, cnt2, wh1h, wh1m, bh1.reshape(1, _HID), wh2,
      bh2.reshape(1, _HID))


def _tc_final(h, wo1, bo1, wo2, bo2, wo3, bo3):
    bn = 1000

    def body(h_ref, w1_ref, b1_ref, w2_ref, b2_ref, w3_ref, b3_ref, o_ref):
        t = _silu(jnp.dot(h_ref[...], w1_ref[...],
                          preferred_element_type=jnp.float32) + b1_ref[...])
        t = jax.nn.relu(jnp.dot(t, w2_ref[...],
                                preferred_element_type=jnp.float32)
                        + b2_ref[...])
        o_ref[...] = jnp.dot(t, w3_ref[...],
                             preferred_element_type=jnp.float32) + b3_ref[...]

    return pl.pallas_call(
        body,
        grid=(_N // bn,),
        in_specs=[
            pl.BlockSpec((bn, _HID), lambda i: (i, 0)),
            pl.BlockSpec((_HID, 1024), lambda i: (0, 0)),
            pl.BlockSpec((1, 1024), lambda i: (0, 0)),
            pl.BlockSpec((1024, 1024), lambda i: (0, 0)),
            pl.BlockSpec((1, 1024), lambda i: (0, 0)),
            pl.BlockSpec((1024, _OUT), lambda i: (0, 0)),
            pl.BlockSpec((1, _OUT), lambda i: (0, 0)),
        ],
        out_specs=pl.BlockSpec((bn, _OUT), lambda i: (i, 0)),
        out_shape=jax.ShapeDtypeStruct((_N, _OUT), jnp.float32),
    )(h, wo1, bo1.reshape(1, 1024), wo2, bo2.reshape(1, 1024), wo3,
      bo3.reshape(1, _OUT))


def _ring(n_chunks, start, process):
    start(0, 0)
    start(1, 1)

    def pair(p, carry):
        for s in (0, 1):
            t = 2 * p + s

            @pl.when(t < n_chunks)
            def _():
                process(t, s)

                @pl.when(t + 2 < n_chunks)
                def _():
                    start(t + 2, s)

        return carry

    lax.fori_loop(0, (n_chunks + 1) // 2, pair, 0)


def _sc_gather(a, b, idx_i, idx_j):
    mesh = plsc.VectorSubcoreMesh(core_axis_name="c", subcore_axis_name="s")

    @functools.partial(
        pl.kernel,
        out_type=jax.ShapeDtypeStruct((_EC, _HID), jnp.float32),
        mesh=mesh,
        scratch_types=[
            pltpu.VMEM((_PREF,), jnp.int32),
            pltpu.VMEM((_PREF,), jnp.int32),
            pltpu.VMEM((2, _GC, _PH), jnp.float32),
            pltpu.VMEM((2, _GC, _PH), jnp.float32),
            pltpu.SemaphoreType.DMA,
            pltpu.SemaphoreType.DMA,
            pltpu.SemaphoreType.DMA,
            pltpu.SemaphoreType.DMA,
        ],
    )
    def k(a_hbm, b_hbm, ii_hbm, jj_hbm, g_hbm, ii_v, jj_v, ra_v, rb_v,
          gsem0, gsem1, osem0, osem1):
        cid = lax.axis_index("c")
        sid = lax.axis_index("s")
        wid = sid * _NC + cid
        gsems = (gsem0, gsem1)
        osems = (osem0, osem1)

        base = (wid * _EW) // 8 * 8
        limit = jnp.where(wid == _NW - 1, _EC, ((wid + 1) * _EW) // 8 * 8)
        pref0 = jnp.minimum(base, _EC - _PREF)

        pltpu.sync_copy(ii_hbm.at[pl.ds(pref0, _PREF)], ii_v)
        pltpu.sync_copy(jj_hbm.at[pl.ds(pref0, _PREF)], jj_v)

        def off_of(t):
            return jnp.minimum(base + t * _GC, limit - _GC)

        def start(t, s):
            loff = off_of(t) - pref0
            pltpu.async_copy(a_hbm.at[ii_v.at[pl.ds(loff, _GC)]], ra_v.at[s],
                             gsems[s])
            pltpu.async_copy(b_hbm.at[jj_v.at[pl.ds(loff, _GC)]], rb_v.at[s],
                             gsems[s])

        def wait_out(t, s):
            off = off_of(t)
            pltpu.make_async_copy(
                ra_v.at[s], g_hbm.at[pl.ds(off, _GC), pl.ds(0, _PH)],
                osems[s]).wait()
            pltpu.make_async_copy(
                rb_v.at[s], g_hbm.at[pl.ds(off, _GC), pl.ds(_PH, _PH)],
                osems[s]).wait()

        def process(t, s):
            pltpu.make_async_copy(a_hbm.at[pl.ds(0, _GC)], ra_v.at[s],
                                  gsems[s]).wait()
            pltpu.make_async_copy(b_hbm.at[pl.ds(0, _GC)], rb_v.at[s],
                                  gsems[s]).wait()

            @pl.when(t >= 2)
            def _():
                wait_out(t - 2, s)

            off = off_of(t)
            pltpu.async_copy(ra_v.at[s],
                             g_hbm.at[pl.ds(off, _GC), pl.ds(0, _PH)],
                             osems[s])
            pltpu.async_copy(rb_v.at[s],
                             g_hbm.at[pl.ds(off, _GC), pl.ds(_PH, _PH)],
                             osems[s])

        _ring(_GT, start, process)
        wait_out(_GT - 2, 0)
        wait_out(_GT - 1, 1)

    return k(a, b, idx_i, idx_j)


def _zero_block(z_v, rows):
    def zrow(r, carry):
        zero = jnp.zeros((_LANE,), jnp.float32)
        for kk in range(_FH // _LANE):
            z_v[r, pl.ds(kk * _LANE, _LANE)] = zero
        return carry

    lax.fori_loop(0, rows, zrow, 0)


def _sc_scatter(msg, idx_i, ones_mode):
    mesh = plsc.VectorSubcoreMesh(core_axis_name="c", subcore_axis_name="s")

    @functools.partial(
        pl.kernel,
        out_type=jax.ShapeDtypeStruct((_N, _HID), jnp.float32),
        mesh=mesh,
        scratch_types=[
            pltpu.VMEM((_LANE,), jnp.int32),
            pltpu.VMEM((2, _SCC), jnp.int32),
            pltpu.VMEM((2, _SCC, _FH), jnp.float32),
            pltpu.VMEM((_SCC, _FH), jnp.float32),
            pltpu.VMEM_SHARED((_NPAD, _FH), jnp.float32),
            pltpu.SemaphoreType.DMA,
            pltpu.SemaphoreType.DMA,
        ],
    )
    def k(msg_hbm, ii_hbm, fl_hbm, agg_hbm, fl_v, ii_v, mb_v, z_v, acc_sh,
          sem0, sem1):
        cid = lax.axis_index("c")
        sid = lax.axis_index("s")
        col0 = cid * _FH
        sems = (sem0, sem1)
        row0 = sid * _NR
        c0 = sid * _SNCH // _NS
        c1 = (sid + 1) * _SNCH // _NS
        ntrip = c1 - c0

        pltpu.sync_copy(fl_hbm, fl_v)
        ones = fl_v[pl.ds(0, _LANE)][0] == 1

        _zero_block(z_v, _SCC)
        for zc in range(_NR // _SCC):
            pltpu.sync_copy(z_v, acc_sh.at[pl.ds(row0 + zc * _SCC, _SCC)])
        zrem = _NR - (_NR // _SCC) * _SCC
        if zrem:
            pltpu.sync_copy(
                z_v.at[pl.ds(0, zrem)],
                acc_sh.at[pl.ds(row0 + _NR - zrem, zrem)])
        plsc.subcore_barrier()

        @pl.when(ones)
        def _():
            def orow(r, carry):
                one = jnp.ones((_LANE,), jnp.float32)
                for kk in range(_FH // _LANE):
                    z_v[r, pl.ds(kk * _LANE, _LANE)] = one
                return carry

            lax.fori_loop(0, _SCC, orow, 0)

        def start(t, s):
            off = (c0 + t) * _SCC
            pltpu.async_copy(ii_hbm.at[pl.ds(off, _SCC)], ii_v.at[s], sems[s])

            @pl.when(jnp.logical_not(ones))
            def _():
                pltpu.async_copy(
                    msg_hbm.at[pl.ds(off, _SCC), pl.ds(col0, _FH)],
                    mb_v.at[s], sems[s])

        def process(t, s):
            pltpu.make_async_copy(ii_hbm.at[pl.ds(0, _SCC)], ii_v.at[s],
                                  sems[s]).wait()

            @pl.when(jnp.logical_not(ones))
            def _():
                pltpu.make_async_copy(
                    msg_hbm.at[pl.ds(0, _SCC), pl.ds(col0, _FH)],
                    mb_v.at[s], sems[s]).wait()
                pltpu.sync_copy(mb_v.at[s], acc_sh.at[ii_v.at[s]], add=True)

            @pl.when(ones)
            def _():
                pltpu.sync_copy(z_v, acc_sh.at[ii_v.at[s]], add=True)

        _ring(ntrip, start, process)
        plsc.subcore_barrier()

        @pl.when(sid < _NS - 1)
        def _():
            pltpu.sync_copy(acc_sh.at[pl.ds(row0, _NR)],
                            agg_hbm.at[pl.ds(row0, _NR), pl.ds(col0, _FH)])

        @pl.when(sid == _NS - 1)
        def _():
            pltpu.sync_copy(
                acc_sh.at[pl.ds(row0, _NTAIL)],
                agg_hbm.at[pl.ds(row0, _NTAIL), pl.ds(col0, _FH)])

    flag = jnp.full((_LANE,), 1 if ones_mode else 0, jnp.int32)
    return k(msg, idx_i, flag)


def kernel(h, edge_index, Wm1, bm1, Wm2, bm2, Wa, ba, Wh1, bh1, Wh2, bh2,
           Wo1, bo1, Wo2, bo2, Wo3, bo3):
    idx_i = edge_index[0]
    idx_j = edge_index[1]
    ii = [idx_i[:_EC], idx_i[_EC:]]
    jj = [idx_j[:_EC], idx_j[_EC:]]
    cnt1 = cnt2 = None
    perm = jnp.concatenate([jnp.arange(0, _HID, 2), jnp.arange(1, _HID, 2)])
    for l in range(_L):
        w1a = Wm1[l, :_D]
        w1b = Wm1[l, _D:]
        a, b = _tc_proj(h, w1a[:, 0::2], w1a[:, 1::2], w1b[:, 0::2],
                        w1b[:, 1::2], bm1[l][0::2], bm1[l][1::2])
        wm2p = Wm2[l][perm]
        wa_row = Wa[l].reshape(1, _HID)
        g1 = _sc_gather(a, b, ii[0], jj[0])
        msg1 = _tc_edge(g1, wm2p, bm2[l], wa_row, ba[l])
        g2 = _sc_gather(a, b, ii[1], jj[1])
        msg2 = _tc_edge(g2, wm2p, bm2[l], wa_row, ba[l])
        if l == 0:
            cnt1 = _sc_scatter(msg1, ii[0], ones_mode=True)
            cnt2 = _sc_scatter(msg2, ii[1], ones_mode=True)
        agg1 = _sc_scatter(msg1, ii[0], ones_mode=False)
        agg2 = _sc_scatter(msg2, ii[1], ones_mode=False)
        h = _tc_node(h, agg1, agg2, cnt1, cnt2, Wh1[l, :_D], Wh1[l, _D:],
                     bh1[l], Wh2[l], bh2[l])
    return _tc_final(h, Wo1, bo1, Wo2, bo2, Wo3, bo3)
